# baseline (speedup 1.0000x reference)
import functools

import jax
import jax.numpy as jnp
from jax import lax
from jax.experimental import pallas as pl
from jax.experimental.pallas import tpu as pltpu
from jax.experimental.pallas import tpu_sc as plsc

V = 1000000
B = 16384
H = 20
D = 32
NC = 2
NS = 16
NW = NC * NS

NCOL = V // 128
VTAIL = V - NCOL * 128
COLS_W = NCOL // NW
COL_REM = NCOL - COLS_W * NW

ROWS_W = B // NW
CHUNK = 128
NCHUNK = ROWS_W // CHUNK
IPG = 128
GATHERS = CHUNK * H // IPG
IDX_ROWS_W = ROWS_W * H // IPG


def _relayout_body(tt_hbm, tail_hbm, tflat_hbm, tin, tout, tailv):
    wid = lax.axis_index("s") * NC + lax.axis_index("c")
    start = wid * COLS_W + jnp.minimum(wid, COL_REM)
    cnt = COLS_W + jnp.where(wid < COL_REM, 1, 0)
    iota32 = lax.iota(jnp.int32, 16) * 32

    def col_body(j, carry):
        pltpu.sync_copy(tt_hbm.at[:, pl.ds(j * 128, 128)], tin)
        for g in range(8):
            for d in range(D):
                v = tin[d, pl.ds(g * 16, 16)]
                plsc.store_scatter(tout, [iota32 + (g * 512 + d)], v)
        pltpu.sync_copy(tout, tflat_hbm.at[pl.ds(j * 4096, 4096)])
        return carry

    lax.fori_loop(start, start + cnt, col_body, 0)

    @pl.when(wid == NW - 1)
    def _():
        pltpu.sync_copy(tail_hbm, tailv)
        for r in range(VTAIL):
            for h in range(2):
                tout[pl.ds(r * D + h * 16, 16)] = tailv[r, pl.ds(h * 16, 16)]
        pltpu.sync_copy(tout.at[pl.ds(0, VTAIL * D)],
                        tflat_hbm.at[pl.ds(NCOL * 4096, VTAIL * D)])


_relayout = functools.partial(
    pl.kernel,
    out_type=jax.ShapeDtypeStruct((V * D,), jnp.float32),
    mesh=plsc.VectorSubcoreMesh(core_axis_name="c", subcore_axis_name="s"),
    compiler_params=pltpu.CompilerParams(
        use_tc_tiling_on_sc=True, needs_layout_passes=False),
    scratch_types=[
        pltpu.VMEM((D, 128), jnp.float32),
        pltpu.VMEM((128 * D,), jnp.float32),
        pltpu.VMEM((VTAIL, 128), jnp.float32),
    ],
)(_relayout_body)


def _embed_body(idx_hbm, table_hbm, out_hbm, idx_v, buf, outbuf, sem):
    wid = lax.axis_index("s") * NC + lax.axis_index("c")
    base = wid * ROWS_W

    pltpu.sync_copy(idx_hbm.at[pl.ds(wid * IDX_ROWS_W, IDX_ROWS_W)], idx_v)

    def chunk_body(c, carry):
        copies = []
        for k in range(GATHERS):
            copies.append(pltpu.async_copy(
                table_hbm.at[idx_v.at[c * GATHERS + k]],
                buf.at[pl.ds(k * IPG, IPG)],
                sem))
        for cp in copies:
            cp.wait()

        def red_body(r, carry2):
            for half in range(2):
                sl = pl.ds(half * 16, 16)
                v = buf[r * H, sl]
                for j in range(1, H):
                    v = v + buf[r * H + j, sl]
                outbuf[r, sl] = v * (1.0 / H)
            return carry2

        lax.fori_loop(0, CHUNK, red_body, 0)
        pltpu.sync_copy(outbuf, out_hbm.at[pl.ds(base + c * CHUNK, CHUNK)])
        return carry

    lax.fori_loop(0, NCHUNK, chunk_body, 0)


_embed = functools.partial(
    pl.kernel,
    out_type=jax.ShapeDtypeStruct((B, D), jnp.float32),
    mesh=plsc.VectorSubcoreMesh(core_axis_name="c", subcore_axis_name="s"),
    compiler_params=pltpu.CompilerParams(use_tc_tiling_on_sc=False),
    scratch_types=[
        pltpu.VMEM((B * H // IPG // NW, IPG), jnp.int32),
        pltpu.VMEM((CHUNK * H, D), jnp.float32),
        pltpu.VMEM((CHUNK, D), jnp.float32),
        pltpu.SemaphoreType.DMA,
    ],
)(_embed_body)


def kernel(label_lists, table):
    idx = label_lists.astype(jnp.int32).reshape(B * H // IPG, IPG)
    tailpad = jnp.pad(table[NCOL * 128:, :], ((0, 0), (0, 128 - D)))
    tflat = _relayout(table.T, tailpad)
    return _embed(idx, tflat.reshape(V, D))


# speedup vs baseline: 1.7608x; 1.7608x over previous
import functools

import jax
import jax.numpy as jnp
from jax import lax
from jax.experimental import pallas as pl
from jax.experimental.pallas import tpu as pltpu
from jax.experimental.pallas import tpu_sc as plsc

V = 1000000
B = 16384
H = 20
D = 32
NC = 2
NS = 16
NW = NC * NS

NCOL = V // 128
VTAIL = V - NCOL * 128
COLS_W = NCOL // NW
COL_REM = NCOL - COLS_W * NW

ROWS_W = B // NW
CHUNK = 128
NCHUNK = ROWS_W // CHUNK
IPG = 128
GATHERS = CHUNK * H // IPG
IDX_ROWS_W = ROWS_W * H // IPG


def _relayout_body(tt_hbm, tail_hbm, tflat_hbm, tin0, tin1, tout0, tout1,
                   tailv, in_sem0, in_sem1, out_sem0, out_sem1):
    tins = (tin0, tin1)
    touts = (tout0, tout1)
    wid = lax.axis_index("s") * NC + lax.axis_index("c")
    start = wid * COLS_W + jnp.minimum(wid, 2) * 2
    cnt = COLS_W + jnp.where(wid < 2, 2, 0)
    iota32 = lax.iota(jnp.int32, 16) * 32
    in_sems = (in_sem0, in_sem1)
    out_sems = (out_sem0, out_sem1)

    def col_in(j, s):
        return pltpu.make_async_copy(
            tt_hbm.at[:, pl.ds(j * 128, 128)], tins[s], in_sems[s])

    def col_out(j, s):
        return pltpu.make_async_copy(
            touts[s], tflat_hbm.at[pl.ds(j * 4096, 4096)], out_sems[s])

    col_in(start, 0).start()

    def pair_body(p, carry):
        for s in range(2):
            k = p * 2 + s
            j = start + k
            col_in(j, s).wait()

            @pl.when(k + 1 < cnt)
            def _():
                col_in(j + 1, s ^ 1).start()

            @pl.when(k >= 2)
            def _():
                col_out(j - 2, s).wait()

            @plsc.parallel_loop(0, 128 * D // 16, unroll=8)
            def _(i):
                g = i >> 5
                d = i & 31
                v = tins[s][d, pl.ds(g * 16, 16)]
                plsc.store_scatter(touts[s], [iota32 + (g * 512 + d)], v)

            col_out(j, s).start()
        return carry

    lax.fori_loop(0, cnt // 2, pair_body, 0)
    col_out(start + cnt - 2, 0).wait()
    col_out(start + cnt - 1, 1).wait()

    @pl.when(wid == NW - 1)
    def _():
        pltpu.sync_copy(tail_hbm, tailv)
        for r in range(VTAIL):
            for h in range(2):
                tout0[pl.ds(r * D + h * 16, 16)] = tailv[r, pl.ds(h * 16, 16)]
        pltpu.sync_copy(tout0.at[pl.ds(0, VTAIL * D)],
                        tflat_hbm.at[pl.ds(NCOL * 4096, VTAIL * D)])


_relayout = functools.partial(
    pl.kernel,
    out_type=jax.ShapeDtypeStruct((V * D,), jnp.float32),
    mesh=plsc.VectorSubcoreMesh(core_axis_name="c", subcore_axis_name="s"),
    compiler_params=pltpu.CompilerParams(
        use_tc_tiling_on_sc=True, needs_layout_passes=False),
    scratch_types=[
        pltpu.VMEM((D, 128), jnp.float32),
        pltpu.VMEM((D, 128), jnp.float32),
        pltpu.VMEM((128 * D,), jnp.float32),
        pltpu.VMEM((128 * D,), jnp.float32),
        pltpu.VMEM((VTAIL, 128), jnp.float32),
        pltpu.SemaphoreType.DMA,
        pltpu.SemaphoreType.DMA,
        pltpu.SemaphoreType.DMA,
        pltpu.SemaphoreType.DMA,
    ],
)(_relayout_body)


def _embed_body(idx_hbm, table_hbm, out_hbm, idx_v, buf, outbuf, sem):
    wid = lax.axis_index("s") * NC + lax.axis_index("c")
    base = wid * ROWS_W

    pltpu.sync_copy(idx_hbm.at[pl.ds(wid * IDX_ROWS_W, IDX_ROWS_W)], idx_v)

    def chunk_body(c, carry):
        copies = []
        for k in range(GATHERS):
            copies.append(pltpu.async_copy(
                table_hbm.at[idx_v.at[c * GATHERS + k]],
                buf.at[pl.ds(k * IPG, IPG)],
                sem))
        for cp in copies:
            cp.wait()

        def red_body(r, carry2):
            for half in range(2):
                sl = pl.ds(half * 16, 16)
                v = buf[r * H, sl]
                for j in range(1, H):
                    v = v + buf[r * H + j, sl]
                outbuf[r, sl] = v * (1.0 / H)
            return carry2

        lax.fori_loop(0, CHUNK, red_body, 0)
        pltpu.sync_copy(outbuf, out_hbm.at[pl.ds(base + c * CHUNK, CHUNK)])
        return carry

    lax.fori_loop(0, NCHUNK, chunk_body, 0)


_embed = functools.partial(
    pl.kernel,
    out_type=jax.ShapeDtypeStruct((B, D), jnp.float32),
    mesh=plsc.VectorSubcoreMesh(core_axis_name="c", subcore_axis_name="s"),
    compiler_params=pltpu.CompilerParams(use_tc_tiling_on_sc=False),
    scratch_types=[
        pltpu.VMEM((B * H // IPG // NW, IPG), jnp.int32),
        pltpu.VMEM((CHUNK * H, D), jnp.float32),
        pltpu.VMEM((CHUNK, D), jnp.float32),
        pltpu.SemaphoreType.DMA,
    ],
)(_embed_body)


def kernel(label_lists, table):
    idx = label_lists.astype(jnp.int32).reshape(B * H // IPG, IPG)
    tailpad = jnp.pad(table[NCOL * 128:, :], ((0, 0), (0, 128 - D)))
    tflat = _relayout(table.T, tailpad)
    return _embed(idx, tflat.reshape(V, D))
